# baseline (speedup 1.0000x reference)
import functools

import jax
import jax.numpy as jnp
from jax import lax
from jax.experimental import pallas as pl
from jax.experimental.pallas import tpu as pltpu
from jax.experimental.pallas import tpu_sc as plsc

F = 128
EPS = 1e-5
NSC = 2
NTILES = 16
CHUNK = 16
NCHUNKS = F // CHUNK
IB = 128
G = 6
EPAD = NTILES * IB * G


@functools.lru_cache(maxsize=None)
def _make_seg_sum(n_src, n_dst, e_pad, split_dst, chunk):
    nchunks = F // chunk
    ept = e_pad // NTILES
    nb = ept // IB
    ng = nb // G
    if split_dst:
        rows = n_dst // NSC
        my_chunks = nchunks
    else:
        rows = n_dst
        my_chunks = nchunks // NSC
    acc_rows = rows + 512
    rpt = rows // NTILES
    azt = acc_rows // NTILES
    ZB = 16384 // chunk

    mesh = plsc.VectorSubcoreMesh(core_axis_name="c", subcore_axis_name="s")

    @functools.partial(
        pl.kernel,
        out_type=jax.ShapeDtypeStruct((n_dst, F), jnp.float32),
        mesh=mesh,
        compiler_params=pltpu.CompilerParams(use_tc_tiling_on_sc=False),
        scratch_types=[
            pltpu.VMEM((nb, IB), jnp.int32),
            pltpu.VMEM((nb, IB), jnp.int32),
            pltpu.VMEM((G, IB, chunk), jnp.float32),
            pltpu.VMEM((ZB, chunk), jnp.float32),
            pltpu.VMEM_SHARED((acc_rows, chunk), jnp.float32),
            pltpu.SemaphoreType.DMA,
        ],
    )
    def seg_sum(table_hbm, src_hbm, dst_hbm, out_hbm,
                src_v, dst_v, gbuf, zbuf, acc, gsem):
        cid = lax.axis_index("c")
        tid = lax.axis_index("s")

        pltpu.sync_copy(src_hbm.at[tid], src_v)
        pltpu.sync_copy(dst_hbm.at[tid], dst_v)

        def zfill(i, _):
            for k in range(chunk // 16):
                zbuf[i, pl.ds(k * 16, 16)] = jnp.zeros((16,), jnp.float32)
            return 0
        lax.fori_loop(0, ZB, zfill, 0)

        if split_dst:
            base = cid * rows
            def remap(b, _):
                for k in range(IB // 16):
                    v = dst_v[b, pl.ds(k * 16, 16)]
                    lv = v - base
                    ok = (lv >= 0) & (lv < rows)
                    dst_v[b, pl.ds(k * 16, 16)] = jnp.where(ok, lv, rows)
                return 0
            lax.fori_loop(0, nb, remap, 0)

        for j in range(my_chunks):
            jg = j if split_dst else cid * my_chunks + j
            r0 = tid * azt
            off = 0
            for _ in range(azt // ZB):
                pltpu.sync_copy(zbuf, acc.at[pl.ds(r0 + off, ZB)])
                off += ZB
            if azt % ZB:
                pltpu.sync_copy(zbuf.at[pl.ds(0, azt % ZB)],
                                acc.at[pl.ds(r0 + off, azt % ZB)])
            plsc.subcore_barrier()

            tbl = table_hbm.at[jg]

            def group(g, _):
                b0 = g * G
                hnds = []
                for k in range(G):
                    hnds.append(pltpu.async_copy(
                        tbl.at[src_v.at[b0 + k]], gbuf.at[k], gsem))
                for k in range(G):
                    hnds[k].wait()
                for k in range(G):
                    pltpu.sync_copy(gbuf.at[k], acc.at[dst_v.at[b0 + k]],
                                    add=True)
                return 0
            lax.fori_loop(0, ng, group, 0)
            plsc.subcore_barrier()

            w0 = tid * rpt
            dst_row = (cid * rows if split_dst else 0) + w0
            pltpu.sync_copy(acc.at[pl.ds(w0, rpt)],
                            out_hbm.at[pl.ds(dst_row, rpt), pl.ds(jg * chunk, chunk)])
            plsc.subcore_barrier()

    return seg_sum


@functools.lru_cache(maxsize=None)
def _make_seg_sum_fullrow(n_src, n_dst, e_pad):
    BF = 64
    ept = e_pad // (NSC * NTILES)
    nb = ept // BF
    acc_rows = n_dst + 512
    rpt = n_dst // NTILES
    azt = acc_rows // NTILES
    ZB = BF

    mesh = plsc.VectorSubcoreMesh(core_axis_name="c", subcore_axis_name="s")

    @functools.partial(
        pl.kernel,
        out_type=jax.ShapeDtypeStruct((NSC, n_dst, F), jnp.float32),
        mesh=mesh,
        compiler_params=pltpu.CompilerParams(use_tc_tiling_on_sc=False),
        scratch_types=[
            pltpu.VMEM((nb, BF), jnp.int32),
            pltpu.VMEM((nb, BF), jnp.int32),
            pltpu.VMEM((3, BF, F), jnp.float32),
            pltpu.VMEM_SHARED((acc_rows, F), jnp.float32),
            pltpu.SemaphoreType.DMA,
        ],
    )
    def seg_sum_fr(table_hbm, src_hbm, dst_hbm, out_hbm,
                   src_v, dst_v, gbuf, acc, gsem):
        cid = lax.axis_index("c")
        tid = lax.axis_index("s")
        wid = cid * NTILES + tid

        pltpu.sync_copy(src_hbm.at[wid], src_v)
        pltpu.sync_copy(dst_hbm.at[wid], dst_v)

        def zfill(i, _):
            for k in range(F // 16):
                gbuf[0, i, pl.ds(k * 16, 16)] = jnp.zeros((16,), jnp.float32)
            return 0
        lax.fori_loop(0, BF, zfill, 0)
        zsrc = gbuf.at[0]
        r0 = tid * azt
        off = 0
        for _ in range(azt // ZB):
            pltpu.sync_copy(zsrc, acc.at[pl.ds(r0 + off, ZB)])
            off += ZB
        if azt % ZB:
            pltpu.sync_copy(zsrc.at[pl.ds(0, azt % ZB)],
                            acc.at[pl.ds(r0 + off, azt % ZB)])
        plsc.subcore_barrier()

        def fire(b, slot):
            pltpu.async_copy(table_hbm.at[src_v.at[b]], gbuf.at[slot], gsem)

        def drain(slot):
            pltpu.make_async_copy(table_hbm.at[src_v.at[0]],
                                  gbuf.at[slot], gsem).wait()

        fire(0, 0)
        fire(1, 1)

        def step(g, _):
            sl = g % 3
            drain(sl)
            fire(g + 2, (g + 2) % 3)
            pltpu.sync_copy(gbuf.at[sl], acc.at[dst_v.at[g]], add=True)
            return 0
        lax.fori_loop(0, nb - 2, step, 0)

        def tail(g, _):
            sl = g % 3
            drain(sl)
            pltpu.sync_copy(gbuf.at[sl], acc.at[dst_v.at[g]], add=True)
            return 0
        lax.fori_loop(nb - 2, nb, tail, 0)
        plsc.subcore_barrier()

        pltpu.sync_copy(acc.at[pl.ds(tid * rpt, rpt)],
                        out_hbm.at[cid].at[pl.ds(tid * rpt, rpt)])

    return seg_sum_fr


NQ = 4


@functools.lru_cache(maxsize=None)
def _make_partition(n_dst, e_pad):
    NW = NSC * NTILES
    npt = e_pad // NW
    nv = npt // 16
    qrows = n_dst // NQ
    capB = npt + 1024

    mesh = plsc.VectorSubcoreMesh(core_axis_name="c", subcore_axis_name="s")
    stage_t = pltpu.VMEM((capB,), jnp.int32)

    @functools.partial(
        pl.kernel,
        out_type=[jax.ShapeDtypeStruct((NW, NQ, capB), jnp.int32),
                  jax.ShapeDtypeStruct((NW, NQ, capB), jnp.int32),
                  jax.ShapeDtypeStruct((NW, NQ, 16), jnp.int32)],
        mesh=mesh,
        compiler_params=pltpu.CompilerParams(use_tc_tiling_on_sc=False),
        scratch_types=[
            pltpu.VMEM((npt,), jnp.int32),
            pltpu.VMEM((npt,), jnp.int32),
            stage_t, stage_t, stage_t, stage_t,
            stage_t, stage_t, stage_t, stage_t,
            pltpu.VMEM((NQ, 16), jnp.int32),
            pltpu.VMEM((1024,), jnp.int32),
            pltpu.VMEM((1024,), jnp.int32),
        ],
    )
    def part(src_hbm, dst_hbm, srcP, dstP, cnts,
             src_all, dst_all, s0, s1, s2, s3, d0, d1, d2, d3,
             cnt_buf, pat_s, pat_d):
        cid = lax.axis_index("c")
        tid = lax.axis_index("s")
        wid = cid * NTILES + tid
        st_s = (s0, s1, s2, s3)
        st_d = (d0, d1, d2, d3)

        pltpu.sync_copy(src_hbm.at[wid], src_all)
        pltpu.sync_copy(dst_hbm.at[wid], dst_all)

        zeros16 = jnp.zeros((16,), jnp.int32)

        iot = lax.iota(jnp.int32, 16)

        def pfill(i, _):
            pat_s[pl.ds(i * 16, 16)] = zeros16
            base = lax.rem(i, 32) * 16
            pat_d[pl.ds(i * 16, 16)] = qrows + base + iot
            return 0
        lax.fori_loop(0, 1024 // 16, pfill, 0)

        def step(v, carry):
            sv = src_all[pl.ds(v * 16, 16)]
            dv = dst_all[pl.ds(v * 16, 16)]
            q = ((dv >= qrows).astype(jnp.int32)
                 + (dv >= 2 * qrows).astype(jnp.int32)
                 + (dv >= 3 * qrows).astype(jnp.int32))
            lv = dv - q * qrows
            new = []
            for qq in range(NQ):
                base = carry[qq]
                m = q == qq
                mi = m.astype(jnp.int32)
                rank = plsc.cumsum(mi) - mi
                pos = base + rank
                plsc.store_scatter(st_s[qq], [pos], sv, mask=m)
                plsc.store_scatter(st_d[qq], [pos], lv, mask=m)
                new.append(base + plsc.all_reduce_population_count(m))
            return tuple(new)

        z = jnp.zeros((16,), jnp.int32)
        carry = lax.fori_loop(0, nv, step, (z, z, z, z))

        for qq in range(NQ):
            base = carry[qq]
            pad = (8 - (base & 7)) & 7
            m = iot < pad
            plsc.store_scatter(st_s[qq], [base + iot], zeros16, mask=m)
            plsc.store_scatter(st_d[qq], [base + iot], qrows + iot, mask=m)
            cnt_buf[qq, :] = base + pad

        for qq in range(NQ):
            c8 = cnt_buf[qq, pl.ds(0, 16)][0]
            pltpu.sync_copy(pat_s, st_s[qq].at[pl.ds(c8, 1024)])
            pltpu.sync_copy(pat_d, st_d[qq].at[pl.ds(c8, 1024)])
            pltpu.sync_copy(cnt_buf.at[qq], cnts.at[wid].at[qq])
            pltpu.sync_copy(st_s[qq], srcP.at[wid].at[qq])
            pltpu.sync_copy(st_d[qq], dstP.at[wid].at[qq])

    return part


GQ = 4


@functools.lru_cache(maxsize=None)
def _make_seg_sum_part(n_src, n_dst, e_pad):
    chunk = 32
    nch = F // chunk
    NW = NSC * NTILES
    npt = e_pad // NW
    capB = npt + 1024
    nbB = capB // IB
    qrows = n_dst // NQ
    acc_rows = qrows + 512
    rpt = qrows // NTILES
    azt = acc_rows // NTILES

    mesh = plsc.VectorSubcoreMesh(core_axis_name="c", subcore_axis_name="s")
    idx_t = pltpu.VMEM((nbB, IB), jnp.int32)

    @functools.partial(
        pl.kernel,
        out_type=jax.ShapeDtypeStruct((n_dst, F), jnp.float32),
        mesh=mesh,
        compiler_params=pltpu.CompilerParams(use_tc_tiling_on_sc=False),
        scratch_types=[
            idx_t, idx_t,
            pltpu.VMEM((3, IB, chunk), jnp.float32),
            pltpu.VMEM((IB, chunk), jnp.float32),
            pltpu.VMEM((NQ, 16), jnp.int32),
            pltpu.VMEM((NQ, 16), jnp.int32),
            pltpu.VMEM_SHARED((acc_rows, chunk), jnp.float32),
            pltpu.SemaphoreType.DMA,
        ],
    )
    def seg(table_hbm, srcP, dstP, cnts, out_hbm,
            sv, dv, gbuf, zbuf, cw0, cw1, acc, gsem):
        cid = lax.axis_index("c")
        tid = lax.axis_index("s")
        w0 = 2 * tid
        w1 = 2 * tid + 1

        pltpu.sync_copy(cnts.at[w0], cw0)
        pltpu.sync_copy(cnts.at[w1], cw1)

        def zfill(i, _):
            for k in range(chunk // 16):
                zbuf[i, pl.ds(k * 16, 16)] = jnp.zeros((16,), jnp.float32)
            return 0
        lax.fori_loop(0, IB, zfill, 0)

        for qi in range(2):
            q = 2 * cid + qi
            cnt0 = cw0[q, pl.ds(0, 16)][0]
            cnt1 = cw1[q, pl.ds(0, 16)][0]
            nb0 = jnp.maximum((cnt0 + IB - 1) // IB, 2)
            nb1 = jnp.maximum((cnt1 + IB - 1) // IB, 2)

            for c in range(nch):
                zsrc = zbuf
                r0 = tid * azt
                off = 0
                for _ in range(azt // IB):
                    pltpu.sync_copy(zsrc, acc.at[pl.ds(r0 + off, IB)])
                    off += IB
                if azt % IB:
                    pltpu.sync_copy(zsrc.at[pl.ds(0, azt % IB)],
                                    acc.at[pl.ds(r0 + off, azt % IB)])
                plsc.subcore_barrier()

                tbl = table_hbm.at[c]
                for w, nbd in ((w0, nb0), (w1, nb1)):
                    pltpu.sync_copy(srcP.at[w].at[q], sv)
                    pltpu.sync_copy(dstP.at[w].at[q], dv)
                    pltpu.async_copy(tbl.at[sv.at[0]], gbuf.at[0], gsem)
                    pltpu.async_copy(tbl.at[sv.at[1]], gbuf.at[1], gsem)

                    def step(g, _):
                        sl = g % 3
                        pltpu.make_async_copy(tbl.at[sv.at[0]],
                                              gbuf.at[sl], gsem).wait()
                        pltpu.async_copy(tbl.at[sv.at[g + 2]],
                                         gbuf.at[(g + 2) % 3], gsem)
                        pltpu.sync_copy(gbuf.at[sl], acc.at[dv.at[g]],
                                        add=True)
                        return 0
                    lax.fori_loop(0, nbd - 2, step, 0)

                    def tail(g, _):
                        sl = g % 3
                        pltpu.make_async_copy(tbl.at[sv.at[0]],
                                              gbuf.at[sl], gsem).wait()
                        pltpu.sync_copy(gbuf.at[sl], acc.at[dv.at[g]],
                                        add=True)
                        return 0
                    lax.fori_loop(nbd - 2, nbd, tail, 0)
                plsc.subcore_barrier()

                pltpu.sync_copy(
                    acc.at[pl.ds(tid * rpt, rpt)],
                    out_hbm.at[pl.ds(q * qrows + tid * rpt, rpt),
                               pl.ds(c * chunk, chunk)])
                plsc.subcore_barrier()

    return seg


def _pad_edges(src, dst, n_dst):
    e = src.shape[0]
    e_pad = -(-e // EPAD) * EPAD
    src_p = jnp.concatenate([src.astype(jnp.int32),
                             jnp.zeros((e_pad - e,), jnp.int32)])
    trash = n_dst + (jnp.arange(e_pad - e, dtype=jnp.int32) % 512)
    dst_p = jnp.concatenate([dst.astype(jnp.int32), trash])
    return src_p, dst_p, e_pad


def _chunk_major(x, chunk):
    n = x.shape[0]
    return x.reshape(n, F // chunk, chunk).transpose(1, 0, 2)


def _seg_sum(x, srcf, dstf, e_pad, n_dst, split_dst, chunk=CHUNK):
    fn = _make_seg_sum(x.shape[0], n_dst, e_pad, split_dst, chunk)
    shape = (NTILES, e_pad // (NTILES * IB), IB)
    return fn(_chunk_major(x, chunk), srcf.reshape(shape), dstf.reshape(shape))


def _seg_sum_fullrow(x, srcf, dstf, e_pad, n_dst):
    fn = _make_seg_sum_fullrow(x.shape[0], n_dst, e_pad)
    shape = (NSC * NTILES, e_pad // (NSC * NTILES * 64), 64)
    return fn(x, srcf.reshape(shape), dstf.reshape(shape))


def _partition(srcf, dstf, e_pad, n_dst):
    nw = NSC * NTILES
    npt = e_pad // nw
    nbb = (npt + 1024) // IB
    sp, dp, cnts = _make_partition(n_dst, e_pad)(
        srcf.reshape(nw, npt), dstf.reshape(nw, npt))
    return (sp.reshape(nw, NQ, nbb, IB), dp.reshape(nw, NQ, nbb, IB), cnts)


def _seg_sum_part(x, sp, dp, cnts, e_pad, n_dst):
    fn = _make_seg_sum_part(x.shape[0], n_dst, e_pad)
    return fn(_chunk_major(x, 32), sp, dp, cnts)


def _tile(n):
    for t in (2000, 1000, 400, 80, 16):
        if n % t == 0:
            return t
    return n


@functools.lru_cache(maxsize=None)
def _make_pass_a(n, up_parts, f_parts):
    t = _tile(n)
    grid = (n // t,)

    def body(*refs):
        idx = 0
        x_ref = refs[idx]; idx += 1
        mu_ref = refs[idx] if up_parts else None
        idx += 1 if up_parts else 0
        mf_ref = refs[idx] if f_parts else None
        idx += 1 if f_parts else 0
        wu_ref, wf_ref, b_ref, yu_ref, yf_ref, st_ref = refs[idx:idx + 6]
        i = pl.program_id(0)
        x = x_ref[...]

        def msg(ref, parts):
            if parts == 2:
                return ref[0] + ref[1]
            return ref[...]
        xu = x + msg(mu_ref, up_parts) if up_parts else x
        xf = x + msg(mf_ref, f_parts) if f_parts else x
        yu = jnp.dot(xu, wu_ref[...], preferred_element_type=jnp.float32) + b_ref[0:1, :]
        yf = jnp.dot(xf, wf_ref[...], preferred_element_type=jnp.float32) + b_ref[1:2, :]
        yu_ref[...] = yu
        yf_ref[...] = yf
        st = jnp.concatenate(
            [yu.sum(0)[None], (yu * yu).sum(0)[None],
             yf.sum(0)[None], (yf * yf).sum(0)[None],
             jnp.zeros((4, F), jnp.float32)], axis=0)

        @pl.when(i == 0)
        def _():
            st_ref[...] = st

        @pl.when(i > 0)
        def _():
            st_ref[...] += st

    row_spec = pl.BlockSpec((t, F), lambda i: (i, 0))
    part_spec = pl.BlockSpec((2, t, F), lambda i: (0, i, 0))
    full_spec = pl.BlockSpec((F, F), lambda i: (0, 0))
    st_spec = pl.BlockSpec((8, F), lambda i: (0, 0))
    in_specs = [row_spec]
    if up_parts:
        in_specs.append(part_spec if up_parts == 2 else row_spec)
    if f_parts:
        in_specs.append(part_spec if f_parts == 2 else row_spec)
    in_specs += [full_spec, full_spec, st_spec]
    return pl.pallas_call(
        body,
        grid=grid,
        in_specs=in_specs,
        out_specs=[row_spec, row_spec, st_spec],
        out_shape=[jax.ShapeDtypeStruct((n, F), jnp.float32),
                   jax.ShapeDtypeStruct((n, F), jnp.float32),
                   jax.ShapeDtypeStruct((8, F), jnp.float32)],
    )


@functools.lru_cache(maxsize=None)
def _make_pass_b(n):
    t = _tile(n)
    grid = (n // t,)
    inv_n = 1.0 / n

    def body(yu_ref, yf_ref, sta_ref, wc_ref, b_ref, yc_ref, st_ref):
        i = pl.program_id(0)
        sta = sta_ref[...]
        mu_u = sta[0:1, :] * inv_n
        su = lax.rsqrt(sta[1:2, :] * inv_n - mu_u * mu_u + EPS)
        mu_f = sta[2:3, :] * inv_n
        sf = lax.rsqrt(sta[3:4, :] * inv_n - mu_f * mu_f + EPS)
        hu = jnp.maximum((yu_ref[...] - mu_u) * su, 0.0)
        hf = jnp.maximum((yf_ref[...] - mu_f) * sf, 0.0)
        yc = (jnp.dot(hu, wc_ref[0:F, :], preferred_element_type=jnp.float32)
              + jnp.dot(hf, wc_ref[F:2 * F, :], preferred_element_type=jnp.float32)
              + b_ref[0:1, :])
        yc_ref[...] = yc
        st = jnp.concatenate(
            [yc.sum(0)[None], (yc * yc).sum(0)[None],
             jnp.zeros((6, F), jnp.float32)], axis=0)

        @pl.when(i == 0)
        def _():
            st_ref[...] = st

        @pl.when(i > 0)
        def _():
            st_ref[...] += st

    row_spec = pl.BlockSpec((t, F), lambda i: (i, 0))
    st_spec = pl.BlockSpec((8, F), lambda i: (0, 0))
    return pl.pallas_call(
        body,
        grid=grid,
        in_specs=[row_spec, row_spec, st_spec,
                  pl.BlockSpec((2 * F, F), lambda i: (0, 0)), st_spec],
        out_specs=[row_spec, st_spec],
        out_shape=[jax.ShapeDtypeStruct((n, F), jnp.float32),
                   jax.ShapeDtypeStruct((8, F), jnp.float32)],
    )


@functools.lru_cache(maxsize=None)
def _make_pass_c(n, readout):
    t = _tile(n)
    grid = (n // t,)
    inv_n = 1.0 / n

    def body(*refs):
        if readout:
            yc_ref, st_ref, w_ref, b_ref, o_ref = refs
        else:
            yc_ref, st_ref, o_ref = refs
        st = st_ref[...]
        m = st[0:1, :] * inv_n
        s = lax.rsqrt(st[1:2, :] * inv_n - m * m + EPS)
        xn = jnp.maximum((yc_ref[...] - m) * s, 0.0)
        if readout:
            o_ref[...] = (jnp.dot(xn, w_ref[...], preferred_element_type=jnp.float32)
                          + b_ref[0:1, :])
        else:
            o_ref[...] = xn

    row_spec = pl.BlockSpec((t, F), lambda i: (i, 0))
    st_spec = pl.BlockSpec((8, F), lambda i: (0, 0))
    in_specs = [row_spec, st_spec]
    if readout:
        in_specs += [pl.BlockSpec((F, F), lambda i: (0, 0)), st_spec]
    return pl.pallas_call(
        body,
        grid=grid,
        in_specs=in_specs,
        out_specs=row_spec,
        out_shape=jax.ShapeDtypeStruct((n, F), jnp.float32),
    )


L = 3


def kernel(x0, x1, x2, params, up_index0, up_index1,
           face_src1, face_dst1, face_src2, face_dst2):
    n0, n1, n2 = x0.shape[0], x1.shape[0], x2.shape[0]

    u0s, u0d, e0 = _pad_edges(up_index0[0], up_index0[1], n0)
    u1s, u1d, e1 = _pad_edges(up_index1[0], up_index1[1], n1)
    f1s, f1d, ef1 = _pad_edges(face_src1, face_dst1, n1)
    f2s, f2d, ef2 = _pad_edges(face_src2, face_dst2, n2)

    def pack_bias(*bs):
        b = jnp.zeros((8, F), jnp.float32)
        for r, v in enumerate(bs):
            b = b.at[r, :].set(v)
        return b

    u1sp, u1dp, u1c = _partition(u1s, u1d, e1, n1)
    f1sp, f1dp, f1c = _partition(f1s, f1d, ef1, n1)

    xs = [x0, x1, x2]
    ns = [n0, n1, n2]
    for l in range(L):
        last = l == L - 1
        m_up0 = _seg_sum_fullrow(xs[0], u0s, u0d, e0, n0)
        if not last:
            m_up1 = _seg_sum_part(xs[1], u1sp, u1dp, u1c, e1, n1)
            m_f1 = _seg_sum_part(xs[0], f1sp, f1dp, f1c, ef1, n1)
            m_f2 = _seg_sum(xs[1], f2s, f2d, ef2, n2, split_dst=False,
                            chunk=32)
            msgs = [(m_up0, None), (m_up1, m_f1), (None, m_f2)]
            dims = (0, 1, 2)
        else:
            msgs = [(m_up0, None)]
            dims = (0,)

        new_xs = list(xs)
        for d in dims:
            n = ns[d]
            mu, mf = msgs[d]
            wu = params[f'W_up_{l}_{d}']
            wf = params[f'W_f_{l}_{d}']
            wc = params[f'W_c_{l}_{d}']
            bab = pack_bias(params[f'b_up_{l}_{d}'], params[f'b_f_{l}_{d}'])
            bcb = pack_bias(params[f'b_c_{l}_{d}'])

            def parts(m):
                if m is None:
                    return 0
                return 2 if m.ndim == 3 else 1
            pa = _make_pass_a(n, parts(mu), parts(mf))
            args = [xs[d]]
            if mu is not None:
                args.append(mu)
            if mf is not None:
                args.append(mf)
            yu, yf, sta = pa(*args, wu, wf, bab)
            yc, stb = _make_pass_b(n)(yu, yf, sta, wc, bcb)
            if last and d == 0:
                w_out = jnp.zeros((F, F), jnp.float32).at[:, :10].set(
                    params['W_out'])
                b_out = pack_bias(jnp.pad(params['b_out'], (0, F - 10)))
                out = _make_pass_c(n, True)(yc, stb, w_out, b_out)
                return out[:, :10]
            new_xs[d] = _make_pass_c(n, False)(yc, stb)
        xs = new_xs


# speedup vs baseline: 4.2813x; 4.2813x over previous
import functools

import jax
import jax.numpy as jnp
from jax import lax
from jax.experimental import pallas as pl
from jax.experimental.pallas import tpu as pltpu
from jax.experimental.pallas import tpu_sc as plsc

F = 128
EPS = 1e-5
NSC = 2
NTILES = 16
CHUNK = 16
NCHUNKS = F // CHUNK
IB = 128
G = 6
EPAD = NTILES * IB * G


@functools.lru_cache(maxsize=None)
def _make_seg_sum(n_src, n_dst, e_pad, split_dst, chunk):
    nchunks = F // chunk
    ept = e_pad // NTILES
    nb = ept // IB
    ng = nb // G
    if split_dst:
        rows = n_dst // NSC
        my_chunks = nchunks
    else:
        rows = n_dst
        my_chunks = nchunks // NSC
    acc_rows = rows + 512
    rpt = rows // NTILES
    azt = acc_rows // NTILES
    ZB = 16384 // chunk

    mesh = plsc.VectorSubcoreMesh(core_axis_name="c", subcore_axis_name="s")

    @functools.partial(
        pl.kernel,
        out_type=jax.ShapeDtypeStruct((n_dst, F), jnp.float32),
        mesh=mesh,
        compiler_params=pltpu.CompilerParams(use_tc_tiling_on_sc=False),
        scratch_types=[
            pltpu.VMEM((nb, IB), jnp.int32),
            pltpu.VMEM((nb, IB), jnp.int32),
            pltpu.VMEM((G, IB, chunk), jnp.float32),
            pltpu.VMEM((ZB, chunk), jnp.float32),
            pltpu.VMEM_SHARED((acc_rows, chunk), jnp.float32),
            pltpu.SemaphoreType.DMA,
        ],
    )
    def seg_sum(table_hbm, src_hbm, dst_hbm, out_hbm,
                src_v, dst_v, gbuf, zbuf, acc, gsem):
        cid = lax.axis_index("c")
        tid = lax.axis_index("s")

        pltpu.sync_copy(src_hbm.at[tid], src_v)
        pltpu.sync_copy(dst_hbm.at[tid], dst_v)

        def zfill(i, _):
            for k in range(chunk // 16):
                zbuf[i, pl.ds(k * 16, 16)] = jnp.zeros((16,), jnp.float32)
            return 0
        lax.fori_loop(0, ZB, zfill, 0)

        if split_dst:
            base = cid * rows
            def remap(b, _):
                for k in range(IB // 16):
                    v = dst_v[b, pl.ds(k * 16, 16)]
                    lv = v - base
                    ok = (lv >= 0) & (lv < rows)
                    dst_v[b, pl.ds(k * 16, 16)] = jnp.where(ok, lv, rows)
                return 0
            lax.fori_loop(0, nb, remap, 0)

        for j in range(my_chunks):
            jg = j if split_dst else cid * my_chunks + j
            r0 = tid * azt
            off = 0
            for _ in range(azt // ZB):
                pltpu.sync_copy(zbuf, acc.at[pl.ds(r0 + off, ZB)])
                off += ZB
            if azt % ZB:
                pltpu.sync_copy(zbuf.at[pl.ds(0, azt % ZB)],
                                acc.at[pl.ds(r0 + off, azt % ZB)])
            plsc.subcore_barrier()

            tbl = table_hbm.at[jg]

            def group(g, _):
                b0 = g * G
                hnds = []
                for k in range(G):
                    hnds.append(pltpu.async_copy(
                        tbl.at[src_v.at[b0 + k]], gbuf.at[k], gsem))
                for k in range(G):
                    hnds[k].wait()
                for k in range(G):
                    pltpu.sync_copy(gbuf.at[k], acc.at[dst_v.at[b0 + k]],
                                    add=True)
                return 0
            lax.fori_loop(0, ng, group, 0)
            plsc.subcore_barrier()

            w0 = tid * rpt
            dst_row = (cid * rows if split_dst else 0) + w0
            pltpu.sync_copy(acc.at[pl.ds(w0, rpt)],
                            out_hbm.at[pl.ds(dst_row, rpt), pl.ds(jg * chunk, chunk)])
            plsc.subcore_barrier()

    return seg_sum


@functools.lru_cache(maxsize=None)
def _make_seg_sum_fullrow(n_src, n_dst, e_pad):
    BF = 64
    ept = e_pad // (NSC * NTILES)
    nb = ept // BF
    acc_rows = n_dst + 512
    rpt = n_dst // NTILES
    azt = acc_rows // NTILES
    ZB = BF

    mesh = plsc.VectorSubcoreMesh(core_axis_name="c", subcore_axis_name="s")

    @functools.partial(
        pl.kernel,
        out_type=jax.ShapeDtypeStruct((NSC, n_dst, F), jnp.float32),
        mesh=mesh,
        compiler_params=pltpu.CompilerParams(use_tc_tiling_on_sc=False),
        scratch_types=[
            pltpu.VMEM((nb, BF), jnp.int32),
            pltpu.VMEM((nb, BF), jnp.int32),
            pltpu.VMEM((3, BF, F), jnp.float32),
            pltpu.VMEM_SHARED((acc_rows, F), jnp.float32),
            pltpu.SemaphoreType.DMA,
        ],
    )
    def seg_sum_fr(table_hbm, src_hbm, dst_hbm, out_hbm,
                   src_v, dst_v, gbuf, acc, gsem):
        cid = lax.axis_index("c")
        tid = lax.axis_index("s")
        wid = cid * NTILES + tid

        pltpu.sync_copy(src_hbm.at[wid], src_v)
        pltpu.sync_copy(dst_hbm.at[wid], dst_v)

        def zfill(i, _):
            for k in range(F // 16):
                gbuf[0, i, pl.ds(k * 16, 16)] = jnp.zeros((16,), jnp.float32)
            return 0
        lax.fori_loop(0, BF, zfill, 0)
        zsrc = gbuf.at[0]
        r0 = tid * azt
        off = 0
        for _ in range(azt // ZB):
            pltpu.sync_copy(zsrc, acc.at[pl.ds(r0 + off, ZB)])
            off += ZB
        if azt % ZB:
            pltpu.sync_copy(zsrc.at[pl.ds(0, azt % ZB)],
                            acc.at[pl.ds(r0 + off, azt % ZB)])
        plsc.subcore_barrier()

        def fire(b, slot):
            pltpu.async_copy(table_hbm.at[src_v.at[b]], gbuf.at[slot], gsem)

        def drain(slot):
            pltpu.make_async_copy(table_hbm.at[src_v.at[0]],
                                  gbuf.at[slot], gsem).wait()

        fire(0, 0)
        fire(1, 1)

        def step(g, _):
            sl = g % 3
            drain(sl)
            fire(g + 2, (g + 2) % 3)
            pltpu.sync_copy(gbuf.at[sl], acc.at[dst_v.at[g]], add=True)
            return 0
        lax.fori_loop(0, nb - 2, step, 0)

        def tail(g, _):
            sl = g % 3
            drain(sl)
            pltpu.sync_copy(gbuf.at[sl], acc.at[dst_v.at[g]], add=True)
            return 0
        lax.fori_loop(nb - 2, nb, tail, 0)
        plsc.subcore_barrier()

        pltpu.sync_copy(acc.at[pl.ds(tid * rpt, rpt)],
                        out_hbm.at[cid].at[pl.ds(tid * rpt, rpt)])

    return seg_sum_fr


NQ = 4


@functools.lru_cache(maxsize=None)
def _make_partition(n_dst, e_pad):
    NW = NSC * NTILES
    npt = e_pad // NW
    nv = npt // 16
    qrows = n_dst // NQ
    capB = npt + 1024

    mesh = plsc.VectorSubcoreMesh(core_axis_name="c", subcore_axis_name="s")
    stage_t = pltpu.VMEM((capB,), jnp.int32)

    @functools.partial(
        pl.kernel,
        out_type=[jax.ShapeDtypeStruct((NW, NQ, capB), jnp.int32),
                  jax.ShapeDtypeStruct((NW, NQ, capB), jnp.int32),
                  jax.ShapeDtypeStruct((NW, NQ, 16), jnp.int32)],
        mesh=mesh,
        compiler_params=pltpu.CompilerParams(use_tc_tiling_on_sc=False),
        scratch_types=[
            pltpu.VMEM((npt,), jnp.int32),
            pltpu.VMEM((npt,), jnp.int32),
            stage_t, stage_t, stage_t, stage_t,
            stage_t, stage_t, stage_t, stage_t,
            pltpu.VMEM((NQ, 16), jnp.int32),
            pltpu.VMEM((1024,), jnp.int32),
            pltpu.VMEM((1024,), jnp.int32),
        ],
    )
    def part(src_hbm, dst_hbm, srcP, dstP, cnts,
             src_all, dst_all, s0, s1, s2, s3, d0, d1, d2, d3,
             cnt_buf, pat_s, pat_d):
        cid = lax.axis_index("c")
        tid = lax.axis_index("s")
        wid = cid * NTILES + tid
        st_s = (s0, s1, s2, s3)
        st_d = (d0, d1, d2, d3)

        pltpu.sync_copy(src_hbm.at[wid], src_all)
        pltpu.sync_copy(dst_hbm.at[wid], dst_all)

        zeros16 = jnp.zeros((16,), jnp.int32)

        iot = lax.iota(jnp.int32, 16)

        def pfill(i, _):
            pat_s[pl.ds(i * 16, 16)] = i * 16 + iot
            base = lax.rem(i, 32) * 16
            pat_d[pl.ds(i * 16, 16)] = qrows + base + iot
            return 0
        lax.fori_loop(0, 1024 // 16, pfill, 0)

        def step(v, carry):
            sv = src_all[pl.ds(v * 16, 16)]
            dv = dst_all[pl.ds(v * 16, 16)]
            q = ((dv >= qrows).astype(jnp.int32)
                 + (dv >= 2 * qrows).astype(jnp.int32)
                 + (dv >= 3 * qrows).astype(jnp.int32))
            lv = dv - q * qrows
            new = []
            for qq in range(NQ):
                base = carry[qq]
                m = q == qq
                mi = m.astype(jnp.int32)
                rank = plsc.cumsum(mi) - mi
                pos = base + rank
                plsc.store_scatter(st_s[qq], [pos], sv, mask=m)
                plsc.store_scatter(st_d[qq], [pos], lv, mask=m)
                new.append(base + plsc.all_reduce_population_count(m))
            return tuple(new)

        z = jnp.zeros((16,), jnp.int32)
        carry = lax.fori_loop(0, nv, step, (z, z, z, z))

        for qq in range(NQ):
            base = carry[qq]
            pad = (8 - (base & 7)) & 7
            m = iot < pad
            plsc.store_scatter(st_s[qq], [base + iot], iot, mask=m)
            plsc.store_scatter(st_d[qq], [base + iot], qrows + iot, mask=m)
            cnt_buf[qq, :] = base + pad

        for qq in range(NQ):
            c8 = cnt_buf[qq, pl.ds(0, 16)][0]
            pltpu.sync_copy(pat_s, st_s[qq].at[pl.ds(c8, 1024)])
            pltpu.sync_copy(pat_d, st_d[qq].at[pl.ds(c8, 1024)])
            pltpu.sync_copy(cnt_buf.at[qq], cnts.at[wid].at[qq])
            pltpu.sync_copy(st_s[qq], srcP.at[wid].at[qq])
            pltpu.sync_copy(st_d[qq], dstP.at[wid].at[qq])

    return part


GQ = 4


@functools.lru_cache(maxsize=None)
def _make_seg_sum_part(n_src, n_dst, e_pad):
    chunk = 32
    nch = F // chunk
    NW = NSC * NTILES
    npt = e_pad // NW
    capB = npt + 1024
    nbB = capB // IB
    qrows = n_dst // NQ
    acc_rows = qrows + 512
    rpt = qrows // NTILES
    azt = acc_rows // NTILES

    mesh = plsc.VectorSubcoreMesh(core_axis_name="c", subcore_axis_name="s")
    idx_t = pltpu.VMEM((nbB, IB), jnp.int32)

    @functools.partial(
        pl.kernel,
        out_type=jax.ShapeDtypeStruct((n_dst, F), jnp.float32),
        mesh=mesh,
        compiler_params=pltpu.CompilerParams(use_tc_tiling_on_sc=False),
        scratch_types=[
            idx_t, idx_t,
            pltpu.VMEM((3, IB, chunk), jnp.float32),
            pltpu.VMEM((IB, chunk), jnp.float32),
            pltpu.VMEM((NQ, 16), jnp.int32),
            pltpu.VMEM((NQ, 16), jnp.int32),
            pltpu.VMEM_SHARED((acc_rows, chunk), jnp.float32),
            pltpu.SemaphoreType.DMA,
        ],
    )
    def seg(table_hbm, srcP, dstP, cnts, out_hbm,
            sv, dv, gbuf, zbuf, cw0, cw1, acc, gsem):
        cid = lax.axis_index("c")
        tid = lax.axis_index("s")
        w0 = 2 * tid
        w1 = 2 * tid + 1

        pltpu.sync_copy(cnts.at[w0], cw0)
        pltpu.sync_copy(cnts.at[w1], cw1)

        def zfill(i, _):
            for k in range(chunk // 16):
                zbuf[i, pl.ds(k * 16, 16)] = jnp.zeros((16,), jnp.float32)
            return 0
        lax.fori_loop(0, IB, zfill, 0)

        for qi in range(2):
            q = 2 * cid + qi
            cnt0 = cw0[q, pl.ds(0, 16)][0]
            cnt1 = cw1[q, pl.ds(0, 16)][0]
            nb0 = jnp.maximum((cnt0 + IB - 1) // IB, 2)
            nb1 = jnp.maximum((cnt1 + IB - 1) // IB, 2)

            for c in range(nch):
                zsrc = zbuf
                r0 = tid * azt
                off = 0
                for _ in range(azt // IB):
                    pltpu.sync_copy(zsrc, acc.at[pl.ds(r0 + off, IB)])
                    off += IB
                if azt % IB:
                    pltpu.sync_copy(zsrc.at[pl.ds(0, azt % IB)],
                                    acc.at[pl.ds(r0 + off, azt % IB)])
                plsc.subcore_barrier()

                tbl = table_hbm.at[c]
                for w, nbd in ((w0, nb0), (w1, nb1)):
                    pltpu.sync_copy(srcP.at[w].at[q], sv)
                    pltpu.sync_copy(dstP.at[w].at[q], dv)
                    pltpu.async_copy(tbl.at[sv.at[0]], gbuf.at[0], gsem)
                    pltpu.async_copy(tbl.at[sv.at[1]], gbuf.at[1], gsem)

                    def step(g, _):
                        sl = g % 3
                        pltpu.make_async_copy(tbl.at[sv.at[0]],
                                              gbuf.at[sl], gsem).wait()
                        pltpu.async_copy(tbl.at[sv.at[g + 2]],
                                         gbuf.at[(g + 2) % 3], gsem)
                        pltpu.sync_copy(gbuf.at[sl], acc.at[dv.at[g]],
                                        add=True)
                        return 0
                    lax.fori_loop(0, nbd - 2, step, 0)

                    def tail(g, _):
                        sl = g % 3
                        pltpu.make_async_copy(tbl.at[sv.at[0]],
                                              gbuf.at[sl], gsem).wait()
                        pltpu.sync_copy(gbuf.at[sl], acc.at[dv.at[g]],
                                        add=True)
                        return 0
                    lax.fori_loop(nbd - 2, nbd, tail, 0)
                plsc.subcore_barrier()

                pltpu.sync_copy(
                    acc.at[pl.ds(tid * rpt, rpt)],
                    out_hbm.at[pl.ds(q * qrows + tid * rpt, rpt),
                               pl.ds(c * chunk, chunk)])
                plsc.subcore_barrier()

    return seg


def _pad_edges(src, dst, n_dst, n_src):
    e = src.shape[0]
    e_pad = -(-e // EPAD) * EPAD
    r = jnp.arange(e_pad - e, dtype=jnp.int32)
    src_p = jnp.concatenate([src.astype(jnp.int32), r % n_src])
    dst_p = jnp.concatenate([dst.astype(jnp.int32), n_dst + (r % 512)])
    return src_p, dst_p, e_pad


def _chunk_major(x, chunk):
    n = x.shape[0]
    return x.reshape(n, F // chunk, chunk).transpose(1, 0, 2)


def _seg_sum(x, srcf, dstf, e_pad, n_dst, split_dst, chunk=CHUNK):
    fn = _make_seg_sum(x.shape[0], n_dst, e_pad, split_dst, chunk)
    shape = (NTILES, e_pad // (NTILES * IB), IB)
    return fn(_chunk_major(x, chunk), srcf.reshape(shape), dstf.reshape(shape))


def _seg_sum_fullrow(x, srcf, dstf, e_pad, n_dst):
    fn = _make_seg_sum_fullrow(x.shape[0], n_dst, e_pad)
    shape = (NSC * NTILES, e_pad // (NSC * NTILES * 64), 64)
    return fn(x, srcf.reshape(shape), dstf.reshape(shape))


def _partition(srcf, dstf, e_pad, n_dst):
    nw = NSC * NTILES
    npt = e_pad // nw
    nbb = (npt + 1024) // IB
    sp, dp, cnts = _make_partition(n_dst, e_pad)(
        srcf.reshape(nw, npt), dstf.reshape(nw, npt))
    return (sp.reshape(nw, NQ, nbb, IB), dp.reshape(nw, NQ, nbb, IB), cnts)


def _seg_sum_part(x, sp, dp, cnts, e_pad, n_dst):
    fn = _make_seg_sum_part(x.shape[0], n_dst, e_pad)
    return fn(_chunk_major(x, 32), sp, dp, cnts)


def _tile(n):
    for t in (2000, 1000, 400, 80, 16):
        if n % t == 0:
            return t
    return n


@functools.lru_cache(maxsize=None)
def _make_pass_a(n, up_parts, f_parts):
    t = _tile(n)
    grid = (n // t,)

    def body(*refs):
        idx = 0
        x_ref = refs[idx]; idx += 1
        mu_ref = refs[idx] if up_parts else None
        idx += 1 if up_parts else 0
        mf_ref = refs[idx] if f_parts else None
        idx += 1 if f_parts else 0
        wu_ref, wf_ref, b_ref, yu_ref, yf_ref, st_ref = refs[idx:idx + 6]
        i = pl.program_id(0)
        x = x_ref[...]

        def msg(ref, parts):
            if parts == 2:
                return ref[0] + ref[1]
            return ref[...]
        xu = x + msg(mu_ref, up_parts) if up_parts else x
        xf = x + msg(mf_ref, f_parts) if f_parts else x
        yu = jnp.dot(xu, wu_ref[...], preferred_element_type=jnp.float32) + b_ref[0:1, :]
        yf = jnp.dot(xf, wf_ref[...], preferred_element_type=jnp.float32) + b_ref[1:2, :]
        yu_ref[...] = yu
        yf_ref[...] = yf
        st = jnp.concatenate(
            [yu.sum(0)[None], (yu * yu).sum(0)[None],
             yf.sum(0)[None], (yf * yf).sum(0)[None],
             jnp.zeros((4, F), jnp.float32)], axis=0)

        @pl.when(i == 0)
        def _():
            st_ref[...] = st

        @pl.when(i > 0)
        def _():
            st_ref[...] += st

    row_spec = pl.BlockSpec((t, F), lambda i: (i, 0))
    part_spec = pl.BlockSpec((2, t, F), lambda i: (0, i, 0))
    full_spec = pl.BlockSpec((F, F), lambda i: (0, 0))
    st_spec = pl.BlockSpec((8, F), lambda i: (0, 0))
    in_specs = [row_spec]
    if up_parts:
        in_specs.append(part_spec if up_parts == 2 else row_spec)
    if f_parts:
        in_specs.append(part_spec if f_parts == 2 else row_spec)
    in_specs += [full_spec, full_spec, st_spec]
    return pl.pallas_call(
        body,
        grid=grid,
        in_specs=in_specs,
        out_specs=[row_spec, row_spec, st_spec],
        out_shape=[jax.ShapeDtypeStruct((n, F), jnp.float32),
                   jax.ShapeDtypeStruct((n, F), jnp.float32),
                   jax.ShapeDtypeStruct((8, F), jnp.float32)],
    )


@functools.lru_cache(maxsize=None)
def _make_pass_b(n):
    t = _tile(n)
    grid = (n // t,)
    inv_n = 1.0 / n

    def body(yu_ref, yf_ref, sta_ref, wc_ref, b_ref, yc_ref, st_ref):
        i = pl.program_id(0)
        sta = sta_ref[...]
        mu_u = sta[0:1, :] * inv_n
        su = lax.rsqrt(sta[1:2, :] * inv_n - mu_u * mu_u + EPS)
        mu_f = sta[2:3, :] * inv_n
        sf = lax.rsqrt(sta[3:4, :] * inv_n - mu_f * mu_f + EPS)
        hu = jnp.maximum((yu_ref[...] - mu_u) * su, 0.0)
        hf = jnp.maximum((yf_ref[...] - mu_f) * sf, 0.0)
        yc = (jnp.dot(hu, wc_ref[0:F, :], preferred_element_type=jnp.float32)
              + jnp.dot(hf, wc_ref[F:2 * F, :], preferred_element_type=jnp.float32)
              + b_ref[0:1, :])
        yc_ref[...] = yc
        st = jnp.concatenate(
            [yc.sum(0)[None], (yc * yc).sum(0)[None],
             jnp.zeros((6, F), jnp.float32)], axis=0)

        @pl.when(i == 0)
        def _():
            st_ref[...] = st

        @pl.when(i > 0)
        def _():
            st_ref[...] += st

    row_spec = pl.BlockSpec((t, F), lambda i: (i, 0))
    st_spec = pl.BlockSpec((8, F), lambda i: (0, 0))
    return pl.pallas_call(
        body,
        grid=grid,
        in_specs=[row_spec, row_spec, st_spec,
                  pl.BlockSpec((2 * F, F), lambda i: (0, 0)), st_spec],
        out_specs=[row_spec, st_spec],
        out_shape=[jax.ShapeDtypeStruct((n, F), jnp.float32),
                   jax.ShapeDtypeStruct((8, F), jnp.float32)],
    )


@functools.lru_cache(maxsize=None)
def _make_pass_c(n, readout):
    t = _tile(n)
    grid = (n // t,)
    inv_n = 1.0 / n

    def body(*refs):
        if readout:
            yc_ref, st_ref, w_ref, b_ref, o_ref = refs
        else:
            yc_ref, st_ref, o_ref = refs
        st = st_ref[...]
        m = st[0:1, :] * inv_n
        s = lax.rsqrt(st[1:2, :] * inv_n - m * m + EPS)
        xn = jnp.maximum((yc_ref[...] - m) * s, 0.0)
        if readout:
            o_ref[...] = (jnp.dot(xn, w_ref[...], preferred_element_type=jnp.float32)
                          + b_ref[0:1, :])
        else:
            o_ref[...] = xn

    row_spec = pl.BlockSpec((t, F), lambda i: (i, 0))
    st_spec = pl.BlockSpec((8, F), lambda i: (0, 0))
    in_specs = [row_spec, st_spec]
    if readout:
        in_specs += [pl.BlockSpec((F, F), lambda i: (0, 0)), st_spec]
    return pl.pallas_call(
        body,
        grid=grid,
        in_specs=in_specs,
        out_specs=row_spec,
        out_shape=jax.ShapeDtypeStruct((n, F), jnp.float32),
    )


L = 3


def kernel(x0, x1, x2, params, up_index0, up_index1,
           face_src1, face_dst1, face_src2, face_dst2):
    n0, n1, n2 = x0.shape[0], x1.shape[0], x2.shape[0]

    u0s, u0d, e0 = _pad_edges(up_index0[0], up_index0[1], n0, n0)
    u1s, u1d, e1 = _pad_edges(up_index1[0], up_index1[1], n1, n1)
    f1s, f1d, ef1 = _pad_edges(face_src1, face_dst1, n1, n0)
    f2s, f2d, ef2 = _pad_edges(face_src2, face_dst2, n2, n1)

    def pack_bias(*bs):
        b = jnp.zeros((8, F), jnp.float32)
        for r, v in enumerate(bs):
            b = b.at[r, :].set(v)
        return b

    u1sp, u1dp, u1c = _partition(u1s, u1d, e1, n1)
    f1sp, f1dp, f1c = _partition(f1s, f1d, ef1, n1)

    xs = [x0, x1, x2]
    ns = [n0, n1, n2]
    for l in range(L):
        last = l == L - 1
        m_up0 = _seg_sum_fullrow(xs[0], u0s, u0d, e0, n0)
        if not last:
            m_up1 = _seg_sum_part(xs[1], u1sp, u1dp, u1c, e1, n1)
            m_f1 = _seg_sum_part(xs[0], f1sp, f1dp, f1c, ef1, n1)
            m_f2 = _seg_sum(xs[1], f2s, f2d, ef2, n2, split_dst=False,
                            chunk=32)
            msgs = [(m_up0, None), (m_up1, m_f1), (None, m_f2)]
            dims = (0, 1, 2)
        else:
            msgs = [(m_up0, None)]
            dims = (0,)

        new_xs = list(xs)
        for d in dims:
            n = ns[d]
            mu, mf = msgs[d]
            wu = params[f'W_up_{l}_{d}']
            wf = params[f'W_f_{l}_{d}']
            wc = params[f'W_c_{l}_{d}']
            bab = pack_bias(params[f'b_up_{l}_{d}'], params[f'b_f_{l}_{d}'])
            bcb = pack_bias(params[f'b_c_{l}_{d}'])

            def parts(m):
                if m is None:
                    return 0
                return 2 if m.ndim == 3 else 1
            pa = _make_pass_a(n, parts(mu), parts(mf))
            args = [xs[d]]
            if mu is not None:
                args.append(mu)
            if mf is not None:
                args.append(mf)
            yu, yf, sta = pa(*args, wu, wf, bab)
            yc, stb = _make_pass_b(n)(yu, yf, sta, wc, bcb)
            if last and d == 0:
                w_out = jnp.zeros((F, F), jnp.float32).at[:, :10].set(
                    params['W_out'])
                b_out = pack_bias(jnp.pad(params['b_out'], (0, F - 10)))
                out = _make_pass_c(n, True)(yc, stb, w_out, b_out)
                return out[:, :10]
            new_xs[d] = _make_pass_c(n, False)(yc, stb)
        xs = new_xs
